# Initial kernel scaffold; baseline (speedup 1.0000x reference)
import jax
import jax.numpy as jnp
from jax.experimental import pallas as pl


def kernel(coors, W_e1, b_e1, W_e2, b_e2, W_m1, b_m1, W_m2, b_m2):
    raise NotImplementedError("write your pallas kernel here")


# speedup vs baseline: 1.3602x; 1.3602x over previous
import functools

import jax
import jax.numpy as jnp
from jax.experimental import pallas as pl

B, N, IN_DIM, OUT_DIM, M_DIM = 2, 1024, 3, 6, 16
BI = 256


def _fused_kernel(c_ref, ct_ref, we1_ref, be1_ref, we2_ref, be2_ref,
                  wm1_ref, bm1_ref, wm2_ref, bm2_ref, out_ref):
    c_blk = c_ref[0]
    c_t = ct_ref[0]

    d = jnp.zeros((BI, N), dtype=jnp.float32)
    for k in range(IN_DIM):
        diff = c_blk[:, k:k + 1] - c_t[k:k + 1, :]
        d = d + diff * diff

    w1 = we1_ref[...]
    b1 = be1_ref[...]
    h0 = jax.nn.silu(d * w1[0, 0] + b1[0, 0])
    h1 = jax.nn.silu(d * w1[0, 1] + b1[0, 1])

    w2 = we2_ref[...]
    b2 = be2_ref[...]
    cols = []
    for c in range(M_DIM):
        m_c = jax.nn.silu(h0 * w2[0, c] + h1 * w2[1, c] + b2[0, c])
        cols.append(jnp.sum(m_c, axis=1, keepdims=True))
    msum = jnp.concatenate(cols, axis=1)

    s = jnp.sum(c_t, axis=1)
    rsum = N * c_blk - s[None, :]

    feats = jnp.concatenate([c_blk, msum, rsum], axis=1)
    h2 = jax.nn.silu(
        jnp.dot(feats, wm1_ref[...], preferred_element_type=jnp.float32)
        + bm1_ref[...])
    out = (jnp.dot(h2, wm2_ref[...], preferred_element_type=jnp.float32)
           + bm2_ref[...])
    out_ref[0] = out


@jax.jit
def kernel(coors, W_e1, b_e1, W_e2, b_e2, W_m1, b_m1, W_m2, b_m2):
    coors_t = jnp.transpose(coors, (0, 2, 1))
    full = lambda shape: pl.BlockSpec(shape, lambda b, i: (0,) * len(shape))
    grid = (B, N // BI)
    return pl.pallas_call(
        _fused_kernel,
        grid=grid,
        in_specs=[
            pl.BlockSpec((1, BI, IN_DIM), lambda b, i: (b, i, 0)),
            pl.BlockSpec((1, IN_DIM, N), lambda b, i: (b, 0, 0)),
            full((1, 2)),
            full((1, 2)),
            full((2, M_DIM)),
            full((1, M_DIM)),
            full((2 * IN_DIM + M_DIM, 2 * M_DIM)),
            full((1, 2 * M_DIM)),
            full((2 * M_DIM, OUT_DIM)),
            full((1, OUT_DIM)),
        ],
        out_specs=pl.BlockSpec((1, BI, OUT_DIM), lambda b, i: (b, i, 0)),
        out_shape=jax.ShapeDtypeStruct((B, N, OUT_DIM), jnp.float32),
    )(coors, coors_t, W_e1, b_e1.reshape(1, -1), W_e2, b_e2.reshape(1, -1),
      W_m1, b_m1.reshape(1, -1), W_m2, b_m2.reshape(1, -1))


# speedup vs baseline: 1.5948x; 1.1725x over previous
import functools

import jax
import jax.numpy as jnp
from jax.experimental import pallas as pl

B, N, IN_DIM, OUT_DIM, M_DIM = 2, 1024, 3, 6, 16
BI = 256


def _silu(x):
    t = 0.5 * x
    return t + t * jnp.tanh(t)


def _fused_kernel(c_ref, ct_ref, we1_ref, be1_ref, we2_ref, be2_ref,
                  wm1_ref, bm1_ref, wm2_ref, bm2_ref, out_ref):
    c_blk = c_ref[0]
    c_t = ct_ref[0]

    cc = jnp.dot(c_blk, c_t, preferred_element_type=jnp.float32)
    ni = jnp.sum(c_blk * c_blk, axis=1, keepdims=True)
    nj = jnp.sum(c_t * c_t, axis=0, keepdims=True)
    d = (ni + nj) - 2.0 * cc

    w1 = we1_ref[...]
    b1 = be1_ref[...]
    h0 = _silu(d * w1[0, 0] + b1[0, 0])
    h1 = _silu(d * w1[0, 1] + b1[0, 1])

    w2 = we2_ref[...]
    b2 = be2_ref[...]
    cols = []
    for c in range(M_DIM):
        m_c = _silu(h0 * w2[0, c] + h1 * w2[1, c] + b2[0, c])
        cols.append(jnp.sum(m_c, axis=1, keepdims=True))
    msum = jnp.concatenate(cols, axis=1)

    s = jnp.sum(c_t, axis=1)
    rsum = N * c_blk - s[None, :]

    feats = jnp.concatenate([c_blk, msum, rsum], axis=1)
    h2 = _silu(
        jnp.dot(feats, wm1_ref[...], preferred_element_type=jnp.float32)
        + bm1_ref[...])
    out = (jnp.dot(h2, wm2_ref[...], preferred_element_type=jnp.float32)
           + bm2_ref[...])
    out_ref[0] = out


@jax.jit
def kernel(coors, W_e1, b_e1, W_e2, b_e2, W_m1, b_m1, W_m2, b_m2):
    coors_t = jnp.transpose(coors, (0, 2, 1))
    full = lambda shape: pl.BlockSpec(shape, lambda b, i: (0,) * len(shape))
    grid = (B, N // BI)
    return pl.pallas_call(
        _fused_kernel,
        grid=grid,
        in_specs=[
            pl.BlockSpec((1, BI, IN_DIM), lambda b, i: (b, i, 0)),
            pl.BlockSpec((1, IN_DIM, N), lambda b, i: (b, 0, 0)),
            full((1, 2)),
            full((1, 2)),
            full((2, M_DIM)),
            full((1, M_DIM)),
            full((2 * IN_DIM + M_DIM, 2 * M_DIM)),
            full((1, 2 * M_DIM)),
            full((2 * M_DIM, OUT_DIM)),
            full((1, OUT_DIM)),
        ],
        out_specs=pl.BlockSpec((1, BI, OUT_DIM), lambda b, i: (b, i, 0)),
        out_shape=jax.ShapeDtypeStruct((B, N, OUT_DIM), jnp.float32),
    )(coors, coors_t, W_e1, b_e1.reshape(1, -1), W_e2, b_e2.reshape(1, -1),
      W_m1, b_m1.reshape(1, -1), W_m2, b_m2.reshape(1, -1))
